# baseline (speedup 1.0000x reference)
import functools

import jax
import jax.numpy as jnp
from jax import lax
from jax.experimental import pallas as pl
from jax.experimental.pallas import tpu as pltpu
from jax.experimental.pallas import tpu_sc as plsc

N_ROWS = 32 * 1024
K = 8192
D = 256
COMMIT = 0.25

BR = 4096
BK = 1024
NR = N_ROWS // BR
NK = K // BK

NC, NS = 2, 16
NW = NC * NS
ROWS_PER_W = N_ROWS // NW
CHUNK = 128
NCHUNK = ROWS_PER_W // CHUNK


def _argmin_body(x_ref, en_ref, idx_ref, loss_ref, min_ref, arg_ref, acc_ref):
    r = pl.program_id(0)
    k = pl.program_id(1)

    x = x_ref[...]
    en = en_ref[...]
    mmn = jnp.dot(x, en, preferred_element_type=jnp.float32)
    x2 = jnp.sum(x * x, axis=1, keepdims=True)
    e2 = 0.25 * jnp.sum(en * en, axis=0, keepdims=True)
    dist = x2 + mmn + e2

    lmin = jnp.min(dist, axis=1, keepdims=True)
    iota = lax.broadcasted_iota(jnp.int32, (BR, BK), 1) + k * BK
    larg = jnp.min(jnp.where(dist == lmin, iota, jnp.int32(2**30)),
                   axis=1, keepdims=True)

    @pl.when(k == 0)
    def _init():
        min_ref[...] = lmin
        arg_ref[...] = larg

    @pl.when(k > 0)
    def _update():
        better = lmin < min_ref[...]
        arg_ref[...] = jnp.where(better, larg, arg_ref[...])
        min_ref[...] = jnp.where(better, lmin, min_ref[...])

    @pl.when(jnp.logical_and(r == 0, k == 0))
    def _zero_acc():
        acc_ref[0] = 0.0

    @pl.when(k == NK - 1)
    def _finish_row_block():
        idx_ref[...] = arg_ref[...]
        acc_ref[0] += jnp.sum(min_ref[...])

    @pl.when(jnp.logical_and(r == NR - 1, k == NK - 1))
    def _emit_loss():
        loss_ref[...] = jnp.full((1, 1), acc_ref[0] * (COMMIT / (N_ROWS * D)),
                                 dtype=jnp.float32)


def _tc_argmin(flat_x, emb_t):
    return pl.pallas_call(
        _argmin_body,
        grid=(NR, NK),
        in_specs=[
            pl.BlockSpec((BR, D), lambda r, k: (r, 0)),
            pl.BlockSpec((D, BK), lambda r, k: (0, k)),
        ],
        out_specs=[
            pl.BlockSpec((BR, 1), lambda r, k: (r, 0)),
            pl.BlockSpec((1, 1), lambda r, k: (0, 0)),
        ],
        out_shape=[
            jax.ShapeDtypeStruct((N_ROWS, 1), jnp.int32),
            jax.ShapeDtypeStruct((1, 1), jnp.float32),
        ],
        scratch_shapes=[
            pltpu.VMEM((BR, 1), jnp.float32),
            pltpu.VMEM((BR, 1), jnp.int32),
            pltpu.SMEM((1,), jnp.float32),
        ],
        compiler_params=pltpu.CompilerParams(
            dimension_semantics=("arbitrary", "arbitrary"),
        ),
    )(flat_x, emb_t)


def _sc_body(idx_hbm, emb_hbm, q_out, counts_out,
             idx_v, rows_v, ones_v, zeros_v, counts_sh, sem0, sem1):
    cid = lax.axis_index("c")
    sid = lax.axis_index("s")
    wid = sid * NC + cid
    base = wid * ROWS_PER_W

    pltpu.sync_copy(idx_hbm.at[pl.ds(wid * NCHUNK, NCHUNK)], idx_v)

    def _fill(i, _):
        ones_v[pl.ds(i * 16, 16)] = jnp.ones((16,), jnp.float32)
        zeros_v[pl.ds(i * 16, 16)] = jnp.zeros((16,), jnp.float32)
        return 0
    lax.fori_loop(0, 32, _fill, 0)

    pltpu.sync_copy(zeros_v, counts_sh.at[pl.ds(sid * (K // NS), K // NS)])
    plsc.subcore_barrier()

    sems = (sem0, sem1)

    def _start(j, buf):
        return pltpu.make_async_copy(
            emb_hbm.at[idx_v.at[j]], rows_v.at[buf], sems[buf])

    cp0 = _start(0, 0)
    cp0.start()
    cp1 = _start(1, 1)
    cp1.start()
    for j in range(NCHUNK):
        buf = j % 2
        _start(j, buf).wait()
        pltpu.sync_copy(rows_v.at[buf],
                        q_out.at[pl.ds(base + j * CHUNK, CHUNK)])
        pltpu.sync_copy(ones_v.at[pl.ds(0, CHUNK)],
                        counts_sh.at[idx_v.at[j]], add=True)
        if j + 2 < NCHUNK:
            _start(j + 2, buf).start()

    plsc.subcore_barrier()

    @pl.when(sid == 0)
    def _emit_counts():
        pltpu.sync_copy(counts_sh, counts_out.at[cid])


def _sc_gather_bincount(idx2d, embedding):
    mesh = plsc.VectorSubcoreMesh(core_axis_name="c", subcore_axis_name="s")
    kern = pl.kernel(
        _sc_body,
        out_type=[
            jax.ShapeDtypeStruct((N_ROWS, D), jnp.float32),
            jax.ShapeDtypeStruct((NC, K), jnp.float32),
        ],
        mesh=mesh,
        scratch_types=[
            pltpu.VMEM((NCHUNK, CHUNK), jnp.int32),
            pltpu.VMEM((2, CHUNK, D), jnp.float32),
            pltpu.VMEM((512, ), jnp.float32),
            pltpu.VMEM((512, ), jnp.float32),
            pltpu.VMEM_SHARED((K,), jnp.float32),
            pltpu.SemaphoreType.DMA,
            pltpu.SemaphoreType.DMA,
        ],
    )
    return kern(idx2d, embedding)


def _perplexity_body(c_ref, out_ref):
    counts = jnp.sum(c_ref[...], axis=0, keepdims=True)
    total = jnp.sum(counts)
    probs = counts / (total + 1e-10)
    ent = jnp.sum(probs * jnp.log(probs + 1e-10))
    out_ref[...] = jnp.full((1, 1), jnp.exp(-ent), dtype=jnp.float32)


def _tc_perplexity(counts2):
    return pl.pallas_call(
        _perplexity_body,
        out_shape=jax.ShapeDtypeStruct((1, 1), jnp.float32),
    )(counts2)


@jax.jit
def kernel(inputs, embedding):
    flat = inputs.reshape(N_ROWS, D)
    emb_t_neg2 = -2.0 * embedding.T

    idx2d, loss2d = _tc_argmin(flat, emb_t_neg2)

    idx_rows = idx2d.reshape(N_ROWS // CHUNK, CHUNK)
    quantized_flat, counts2 = _sc_gather_bincount(idx_rows, embedding)

    perp2d = _tc_perplexity(counts2)

    quantized_st = quantized_flat.reshape(inputs.shape)
    indices = idx2d.reshape(inputs.shape[0], inputs.shape[1])
    return (quantized_st, loss2d[0, 0], indices, perp2d[0, 0])


# speedup vs baseline: 1.1159x; 1.1159x over previous
import functools

import jax
import jax.numpy as jnp
from jax import lax
from jax.experimental import pallas as pl
from jax.experimental.pallas import tpu as pltpu
from jax.experimental.pallas import tpu_sc as plsc

N_ROWS = 32 * 1024
K = 8192
D = 256
COMMIT = 0.25

BR = 2048
BK = 4096
NR = N_ROWS // BR
NK = K // BK

NC, NS = 2, 16
NW = NC * NS
ROWS_PER_W = N_ROWS // NW
CHUNK = 128
NCHUNK = ROWS_PER_W // CHUNK


def _argmin_body(x_ref, en_ref, idx_ref, loss_ref, min_ref, arg_ref, acc_ref):
    r = pl.program_id(0)
    k = pl.program_id(1)

    x = x_ref[...]
    en = en_ref[...]
    mmn = jnp.dot(x, en, preferred_element_type=jnp.float32)
    x2 = jnp.sum(x * x, axis=1, keepdims=True)
    e2 = 0.25 * jnp.sum(en * en, axis=0, keepdims=True)
    dist = x2 + mmn + e2

    lmin = jnp.min(dist, axis=1, keepdims=True)
    iota = lax.broadcasted_iota(jnp.int32, (BR, BK), 1) + k * BK
    larg = jnp.min(jnp.where(dist == lmin, iota, jnp.int32(2**30)),
                   axis=1, keepdims=True)

    @pl.when(k == 0)
    def _init():
        min_ref[...] = lmin
        arg_ref[...] = larg

    @pl.when(k > 0)
    def _update():
        better = lmin < min_ref[...]
        arg_ref[...] = jnp.where(better, larg, arg_ref[...])
        min_ref[...] = jnp.where(better, lmin, min_ref[...])

    @pl.when(jnp.logical_and(r == 0, k == 0))
    def _zero_acc():
        acc_ref[0] = 0.0

    @pl.when(k == NK - 1)
    def _finish_row_block():
        idx_ref[...] = arg_ref[...]
        acc_ref[0] += jnp.sum(min_ref[...])

    @pl.when(jnp.logical_and(r == NR - 1, k == NK - 1))
    def _emit_loss():
        loss_ref[...] = jnp.full((1, 1), acc_ref[0] * (COMMIT / (N_ROWS * D)),
                                 dtype=jnp.float32)


def _tc_argmin(flat_x, emb_t):
    return pl.pallas_call(
        _argmin_body,
        grid=(NR, NK),
        in_specs=[
            pl.BlockSpec((BR, D), lambda r, k: (r, 0)),
            pl.BlockSpec((D, BK), lambda r, k: (0, k)),
        ],
        out_specs=[
            pl.BlockSpec((BR, 1), lambda r, k: (r, 0)),
            pl.BlockSpec((1, 1), lambda r, k: (0, 0)),
        ],
        out_shape=[
            jax.ShapeDtypeStruct((N_ROWS, 1), jnp.int32),
            jax.ShapeDtypeStruct((1, 1), jnp.float32),
        ],
        scratch_shapes=[
            pltpu.VMEM((BR, 1), jnp.float32),
            pltpu.VMEM((BR, 1), jnp.int32),
            pltpu.SMEM((1,), jnp.float32),
        ],
        compiler_params=pltpu.CompilerParams(
            dimension_semantics=("arbitrary", "arbitrary"),
        ),
    )(flat_x, emb_t)


def _sc_body(idx_hbm, emb_hbm, q_out, counts_out,
             idx_v, rows_v, ones_v, zeros_v, counts_sh, sem0, sem1):
    cid = lax.axis_index("c")
    sid = lax.axis_index("s")
    wid = sid * NC + cid
    base = wid * ROWS_PER_W

    pltpu.sync_copy(idx_hbm.at[pl.ds(wid * NCHUNK, NCHUNK)], idx_v)

    def _fill(i, _):
        ones_v[pl.ds(i * 16, 16)] = jnp.ones((16,), jnp.float32)
        zeros_v[pl.ds(i * 16, 16)] = jnp.zeros((16,), jnp.float32)
        return 0
    lax.fori_loop(0, 32, _fill, 0)

    pltpu.sync_copy(zeros_v, counts_sh.at[pl.ds(sid * (K // NS), K // NS)])
    plsc.subcore_barrier()

    sems = (sem0, sem1)

    def _start(j, buf):
        return pltpu.make_async_copy(
            emb_hbm.at[idx_v.at[j]], rows_v.at[buf], sems[buf])

    cp0 = _start(0, 0)
    cp0.start()
    cp1 = _start(1, 1)
    cp1.start()
    for j in range(NCHUNK):
        buf = j % 2
        _start(j, buf).wait()
        pltpu.sync_copy(rows_v.at[buf],
                        q_out.at[pl.ds(base + j * CHUNK, CHUNK)])
        pltpu.sync_copy(ones_v.at[pl.ds(0, CHUNK)],
                        counts_sh.at[idx_v.at[j]], add=True)
        if j + 2 < NCHUNK:
            _start(j + 2, buf).start()

    plsc.subcore_barrier()

    @pl.when(sid == 0)
    def _emit_counts():
        pltpu.sync_copy(counts_sh, counts_out.at[cid])


def _sc_gather_bincount(idx2d, embedding):
    mesh = plsc.VectorSubcoreMesh(core_axis_name="c", subcore_axis_name="s")
    kern = pl.kernel(
        _sc_body,
        out_type=[
            jax.ShapeDtypeStruct((N_ROWS, D), jnp.float32),
            jax.ShapeDtypeStruct((NC, K), jnp.float32),
        ],
        mesh=mesh,
        scratch_types=[
            pltpu.VMEM((NCHUNK, CHUNK), jnp.int32),
            pltpu.VMEM((2, CHUNK, D), jnp.float32),
            pltpu.VMEM((512, ), jnp.float32),
            pltpu.VMEM((512, ), jnp.float32),
            pltpu.VMEM_SHARED((K,), jnp.float32),
            pltpu.SemaphoreType.DMA,
            pltpu.SemaphoreType.DMA,
        ],
    )
    return kern(idx2d, embedding)


def _perplexity_body(c_ref, out_ref):
    counts = jnp.sum(c_ref[...], axis=0, keepdims=True)
    total = jnp.sum(counts)
    probs = counts / (total + 1e-10)
    ent = jnp.sum(probs * jnp.log(probs + 1e-10))
    out_ref[...] = jnp.full((1, 1), jnp.exp(-ent), dtype=jnp.float32)


def _tc_perplexity(counts2):
    return pl.pallas_call(
        _perplexity_body,
        out_shape=jax.ShapeDtypeStruct((1, 1), jnp.float32),
    )(counts2)


@jax.jit
def kernel(inputs, embedding):
    flat = inputs.reshape(N_ROWS, D)
    emb_t_neg2 = -2.0 * embedding.T

    idx2d, loss2d = _tc_argmin(flat, emb_t_neg2)

    idx_rows = idx2d.reshape(N_ROWS // CHUNK, CHUNK)
    quantized_flat, counts2 = _sc_gather_bincount(idx_rows, embedding)

    perp2d = _tc_perplexity(counts2)

    quantized_st = quantized_flat.reshape(inputs.shape)
    indices = idx2d.reshape(inputs.shape[0], inputs.shape[1])
    return (quantized_st, loss2d[0, 0], indices, perp2d[0, 0])
